# baseline (speedup 1.0000x reference)
import functools

import jax
import jax.numpy as jnp
from jax import lax
from jax.experimental import pallas as pl
from jax.experimental.pallas import tpu as pltpu
from jax.experimental.pallas import tpu_sc as plsc

N = 10000
NPAD = 10240
E = 320000
F1 = 128
F2 = 64
NW = 32
EPT = E // NW
CH = 80
NCHUNK = EPT // CH
RPT = NPAD // 16

RI = 4
RR = 3
UNROLL = 12

_MESH = dict(core_axis_name="c", subcore_axis_name="s")


def _wid():
    return lax.axis_index("s") * 2 + lax.axis_index("c")


def _zero16(buf, nwords):
    def zb(i, _):
        buf[pl.ds(i * 16, 16)] = jnp.zeros((16,), jnp.float32)
        return 0
    lax.fori_loop(0, nwords, zb, 0)


@functools.partial(
    pl.kernel,
    out_type=jax.ShapeDtypeStruct((2, NPAD), jnp.float32),
    mesh=plsc.VectorSubcoreMesh(**_MESH),
    scratch_types=(
        [pltpu.VMEM((CH,), jnp.int32) for _ in range(RI)]
        + [pltpu.VMEM((CH,), jnp.float32) for _ in range(RI)]
        + [pltpu.VMEM((RPT,), jnp.float32),
           pltpu.VMEM_SHARED((NPAD,), jnp.float32)]
        + [pltpu.SemaphoreType.DMA for _ in range(2 * RI)]
    ),
    compiler_params=pltpu.CompilerParams(
        needs_layout_passes=False, disable_bounds_checks=True),
)
def _deg_kernel(col_hbm, w_hbm, deg_out, *sc):
    cbuf, wbuf = list(sc[0:RI]), list(sc[RI:2 * RI])
    zbuf, deg_sh = sc[2 * RI], sc[2 * RI + 1]
    sem_i = list(sc[2 * RI + 2:2 * RI + 2 + RI])
    sem_s = list(sc[2 * RI + 2 + RI:])
    c = lax.axis_index("c")
    s = lax.axis_index("s")
    base = _wid() * EPT

    def issue_idx(t, ch):
        off = base + ch * CH
        pltpu.async_copy(col_hbm.at[pl.ds(off, CH)], cbuf[t], sem_i[t])
        pltpu.async_copy(w_hbm.at[pl.ds(off, CH)], wbuf[t], sem_i[t])

    def wait_idx(t, ch):
        off = base + ch * CH
        pltpu.make_async_copy(
            col_hbm.at[pl.ds(off, CH)], cbuf[t], sem_i[t]).wait()
        pltpu.make_async_copy(
            w_hbm.at[pl.ds(off, CH)], wbuf[t], sem_i[t]).wait()

    issue_idx(0, 0)
    issue_idx(1, 1)
    _zero16(zbuf, RPT // 16)
    pltpu.sync_copy(zbuf, deg_sh.at[pl.ds(s * RPT, RPT)])
    plsc.subcore_barrier()

    def group(h, _):
        for u in range(4):
            j = h * 4 + u
            t = u % 4

            @pl.when((j >= 2) & (j <= NCHUNK + 1))
            def _():
                t2 = (u - 2) % 4
                pltpu.make_async_copy(
                    wbuf[t2], deg_sh.at[cbuf[t2]], sem_s[t2]).wait()

            @pl.when(j <= NCHUNK - 3)
            def _():
                issue_idx((u + 2) % 4, j + 2)

            @pl.when(j <= NCHUNK - 1)
            def _():
                wait_idx(t, j)
                pltpu.async_copy(wbuf[t], deg_sh.at[cbuf[t]], sem_s[t],
                                 add=True)
        return 0

    lax.fori_loop(0, 32, group, 0)
    plsc.subcore_barrier()
    pltpu.sync_copy(deg_sh.at[pl.ds(s * RPT, RPT)],
                    deg_out.at[c, pl.ds(s * RPT, RPT)])


def _make_prop_kernel(nfeat, with_norm):
    outs = [jax.ShapeDtypeStruct((2, NPAD, nfeat), jnp.float32)]
    if with_norm:
        outs = [jax.ShapeDtypeStruct((E,), jnp.float32)] + outs
    ZR = 40
    scratch = (
        [pltpu.VMEM((CH,), jnp.int32) for _ in range(RI)]
        + [pltpu.VMEM((CH,), jnp.int32) for _ in range(RI)]
        + [pltpu.VMEM((CH,), jnp.float32) for _ in range(RI)]
        + [pltpu.VMEM((CH, nfeat), jnp.float32) for _ in range(RR)]
        + [pltpu.VMEM((ZR, nfeat), jnp.float32),
           pltpu.VMEM_SHARED((NPAD, nfeat), jnp.float32)]
        + [pltpu.SemaphoreType.DMA for _ in range(RI + RR + RR)]
    )
    if with_norm:
        scratch = (
            [pltpu.VMEM((CH,), jnp.float32) for _ in range(RI)]
            + [pltpu.VMEM((NPAD,), jnp.float32)]
            + scratch
            + [pltpu.SemaphoreType.DMA for _ in range(RI)]
        )

    def body(*refs):
        if with_norm:
            (row_hbm, col_hbm, w_hbm, dinv_hbm, xw_hbm,
             norm_out, part_out) = refs[:7]
            wbuf = list(refs[7:7 + RI])
            dinv_v = refs[7 + RI]
            rest = refs[8 + RI:]
        else:
            (row_hbm, col_hbm, norm_hbm, xw_hbm, part_out) = refs[:5]
            rest = refs[5:]
        ridx = list(rest[0:RI])
        cidx = list(rest[RI:2 * RI])
        nbuf = list(rest[2 * RI:3 * RI])
        rows = list(rest[3 * RI:3 * RI + RR])
        zbuf = rest[3 * RI + RR]
        out_sh = rest[3 * RI + RR + 1]
        sems = rest[3 * RI + RR + 2:]
        sem_i = list(sems[0:RI])
        sem_g = list(sems[RI:RI + RR])
        sem_s = list(sems[RI + RR:RI + RR + RR])
        if with_norm:
            sem_n = list(sems[RI + RR + RR:])

        c = lax.axis_index("c")
        s = lax.axis_index("s")
        base = _wid() * EPT

        def issue_idx(t, ch):
            off = base + ch * CH
            pltpu.async_copy(row_hbm.at[pl.ds(off, CH)], ridx[t], sem_i[t])
            pltpu.async_copy(col_hbm.at[pl.ds(off, CH)], cidx[t], sem_i[t])
            if with_norm:
                pltpu.async_copy(w_hbm.at[pl.ds(off, CH)], wbuf[t], sem_i[t])
            else:
                pltpu.async_copy(norm_hbm.at[pl.ds(off, CH)], nbuf[t],
                                 sem_i[t])

        def wait_idx(t, ch):
            off = base + ch * CH
            pltpu.make_async_copy(
                row_hbm.at[pl.ds(off, CH)], ridx[t], sem_i[t]).wait()
            pltpu.make_async_copy(
                col_hbm.at[pl.ds(off, CH)], cidx[t], sem_i[t]).wait()
            if with_norm:
                pltpu.make_async_copy(
                    w_hbm.at[pl.ds(off, CH)], wbuf[t], sem_i[t]).wait()
            else:
                pltpu.make_async_copy(
                    norm_hbm.at[pl.ds(off, CH)], nbuf[t], sem_i[t]).wait()

        issue_idx(0, 0)
        issue_idx(1, 1)
        if with_norm:
            pltpu.async_copy(dinv_hbm, dinv_v, sem_g[1])

        def zz(i, _):
            for f in range(nfeat // 16):
                zbuf[i, pl.ds(f * 16, 16)] = jnp.zeros((16,), jnp.float32)
            return 0

        lax.fori_loop(0, ZR, zz, 0)
        nz = RPT // ZR
        for q in range(nz):
            pltpu.async_copy(zbuf, out_sh.at[pl.ds(s * RPT + q * ZR, ZR)],
                             sem_s[q % RR])
        wait_idx(0, 0)
        if with_norm:
            pltpu.make_async_copy(dinv_hbm, dinv_v, sem_g[1]).wait()
        pltpu.async_copy(xw_hbm.at[ridx[0]], rows[0], sem_g[0])
        for q in range(nz):
            pltpu.make_async_copy(
                zbuf, out_sh.at[pl.ds(s * RPT + q * ZR, ZR)],
                sem_s[q % RR]).wait()
        plsc.subcore_barrier()

        def group(h, _):
            for u in range(UNROLL):
                j = h * UNROLL + u
                ti = u % RI
                tr = u % RR

                @pl.when((j >= 2) & (j <= NCHUNK + 1))
                def _():
                    pltpu.make_async_copy(
                        rows[(u - 2) % RR],
                        out_sh.at[cidx[(u - 2) % RI]],
                        sem_s[(u - 2) % RR]).wait()

                @pl.when(j <= NCHUNK - 3)
                def _():
                    issue_idx((u + 2) % RI, j + 2)

                @pl.when((j >= 0) & (j <= NCHUNK - 2))
                def _():
                    wait_idx((u + 1) % RI, j + 1)
                    pltpu.async_copy(xw_hbm.at[ridx[(u + 1) % RI]],
                                     rows[(u + 1) % RR],
                                     sem_g[(u + 1) % RR])

                if with_norm:
                    @pl.when((j >= 4) & (j <= NCHUNK + 3))
                    def _():
                        off = base + (j - 4) * CH
                        pltpu.make_async_copy(
                            nbuf[ti], norm_out.at[pl.ds(off, CH)],
                            sem_n[ti]).wait()

                    @pl.when(j <= NCHUNK - 1)
                    def _():
                        for g in range(CH // 16):
                            sl = pl.ds(g * 16, 16)
                            dr = plsc.load_gather(dinv_v, [ridx[ti][sl]])
                            dc = plsc.load_gather(dinv_v, [cidx[ti][sl]])
                            nbuf[ti][sl] = dr * wbuf[ti][sl] * dc
                        pltpu.async_copy(
                            nbuf[ti],
                            norm_out.at[pl.ds(base + j * CH, CH)],
                            sem_n[ti])

                @pl.when(j <= NCHUNK - 1)
                def _():
                    pltpu.make_async_copy(
                        xw_hbm.at[ridx[ti]], rows[tr], sem_g[tr]).wait()

                    def sc(g, _):
                        kbase = g * 8
                        for u2 in range(8):
                            k = kbase + u2
                            bc = plsc.load_gather(
                                nbuf[ti], [jnp.full((16,), k, jnp.int32)])
                            for f in range(nfeat // 16):
                                sl = pl.ds(f * 16, 16)
                                rows[tr][k, sl] = rows[tr][k, sl] * bc
                        return 0

                    lax.fori_loop(0, CH // 8, sc, 0)
                    pltpu.async_copy(rows[tr], out_sh.at[cidx[ti]],
                                     sem_s[tr], add=True)
            return 0

        lax.fori_loop(0, (NCHUNK + 7 + UNROLL - 1) // UNROLL, group, 0)
        plsc.subcore_barrier()
        pltpu.sync_copy(out_sh.at[pl.ds(s * RPT, RPT)],
                        part_out.at[c, pl.ds(s * RPT, RPT)])

    return pl.kernel(
        body,
        out_type=outs if with_norm else outs[0],
        mesh=plsc.VectorSubcoreMesh(**_MESH),
        scratch_types=scratch,
        compiler_params=pltpu.CompilerParams(
            needs_layout_passes=False,
            disable_bounds_checks=True,
            use_tc_tiling_on_sc=None if nfeat % 128 == 0 else False,
        ),
    )


_prop1 = _make_prop_kernel(F1, with_norm=True)
_prop2 = _make_prop_kernel(F2, with_norm=False)


def _matmul1_body(x_ref, w_ref, deg_ref, o_ref, dinv_ref, dinv2_ref):
    o_ref[...] = lax.dot_general(
        x_ref[...], w_ref[...], (((1,), (1,)), ((), ())),
        preferred_element_type=jnp.float32)
    deg = deg_ref[0, :] + deg_ref[1, :] + 1.0
    dinv = jnp.where(deg > 0, lax.rsqrt(deg), 0.0)
    dinv_ref[...] = dinv
    dinv2_ref[...] = dinv * dinv


def _mid_body(p0_ref, p1_ref, xw_ref, dinv2_ref, b_ref, w2_ref, o_ref):
    h = p0_ref[...] + p1_ref[...] + dinv2_ref[...] * xw_ref[...] + b_ref[...]
    h = jnp.maximum(h, 0.0)
    o_ref[...] = lax.dot_general(
        h, w2_ref[...], (((1,), (1,)), ((), ())),
        preferred_element_type=jnp.float32)


def _final_body(p0_ref, p1_ref, xw_ref, dinv2_ref, b_ref, o_ref):
    t = p0_ref[...] + p1_ref[...] + dinv2_ref[...] * xw_ref[...] + b_ref[...]
    m = jnp.max(t, axis=1, keepdims=True)
    lse = jnp.log(jnp.sum(jnp.exp(t - m), axis=1, keepdims=True)) + m
    o_ref[...] = t - lse


_ROWB = 1000


def kernel(x, edge_index, edge_weight, W1, b1, W2, b2):
    row = edge_index[0].astype(jnp.int32)
    col = edge_index[1].astype(jnp.int32)
    w = edge_weight.astype(jnp.float32)

    deg_p = _deg_kernel(col, w)

    _DB = NPAD // (N // _ROWB)
    xw1, dinv, dinv2 = pl.pallas_call(
        _matmul1_body,
        grid=(N // _ROWB,),
        in_specs=[pl.BlockSpec((_ROWB, F1), lambda i: (i, 0)),
                  pl.BlockSpec((F1, F1), lambda i: (0, 0)),
                  pl.BlockSpec((2, _DB), lambda i: (0, i))],
        out_specs=[pl.BlockSpec((_ROWB, F1), lambda i: (i, 0)),
                   pl.BlockSpec((_DB,), lambda i: (i,)),
                   pl.BlockSpec((_DB,), lambda i: (i,))],
        out_shape=[jax.ShapeDtypeStruct((N, F1), jnp.float32),
                   jax.ShapeDtypeStruct((NPAD,), jnp.float32),
                   jax.ShapeDtypeStruct((NPAD,), jnp.float32)],
    )(x, W1, deg_p)

    norm, part1 = _prop1(row, col, w, dinv, xw1)

    dinv2c = dinv2[:N, None]
    xw2 = pl.pallas_call(
        _mid_body,
        grid=(N // _ROWB,),
        in_specs=[pl.BlockSpec((_ROWB, F1), lambda i: (i, 0)),
                  pl.BlockSpec((_ROWB, F1), lambda i: (i, 0)),
                  pl.BlockSpec((_ROWB, F1), lambda i: (i, 0)),
                  pl.BlockSpec((_ROWB, 1), lambda i: (i, 0)),
                  pl.BlockSpec((1, F1), lambda i: (0, 0)),
                  pl.BlockSpec((F2, F1), lambda i: (0, 0))],
        out_specs=pl.BlockSpec((_ROWB, F2), lambda i: (i, 0)),
        out_shape=jax.ShapeDtypeStruct((N, F2), jnp.float32),
    )(part1[0, :N], part1[1, :N], xw1, dinv2c, b1[None, :], W2)

    part2 = _prop2(row, col, norm, xw2)

    out = pl.pallas_call(
        _final_body,
        grid=(N // _ROWB,),
        in_specs=[pl.BlockSpec((_ROWB, F2), lambda i: (i, 0)),
                  pl.BlockSpec((_ROWB, F2), lambda i: (i, 0)),
                  pl.BlockSpec((_ROWB, F2), lambda i: (i, 0)),
                  pl.BlockSpec((_ROWB, 1), lambda i: (i, 0)),
                  pl.BlockSpec((1, F2), lambda i: (0, 0))],
        out_specs=pl.BlockSpec((_ROWB, F2), lambda i: (i, 0)),
        out_shape=jax.ShapeDtypeStruct((N, F2), jnp.float32),
    )(part2[0, :N], part2[1, :N], xw2, dinv2c, b2[None, :])

    return out


# speedup vs baseline: 1.0522x; 1.0522x over previous
import functools

import jax
import jax.numpy as jnp
from jax import lax
from jax.experimental import pallas as pl
from jax.experimental.pallas import tpu as pltpu
from jax.experimental.pallas import tpu_sc as plsc

N = 10000
NPAD = 10240
E = 320000
F1 = 128
F2 = 64
NW = 32
EPT = E // NW
CH = 80
NCHUNK = EPT // CH
RPT = NPAD // 16

RI = 4
RR = 3
UNROLL = 12

_MESH = dict(core_axis_name="c", subcore_axis_name="s")


def _wid():
    return lax.axis_index("s") * 2 + lax.axis_index("c")


def _zero16(buf, nwords):
    def zb(i, _):
        buf[pl.ds(i * 16, 16)] = jnp.zeros((16,), jnp.float32)
        return 0
    lax.fori_loop(0, nwords, zb, 0)


@functools.partial(
    pl.kernel,
    out_type=jax.ShapeDtypeStruct((2, NPAD), jnp.float32),
    mesh=plsc.VectorSubcoreMesh(**_MESH),
    scratch_types=(
        [pltpu.VMEM((CH,), jnp.int32) for _ in range(RI)]
        + [pltpu.VMEM((CH,), jnp.float32) for _ in range(RI)]
        + [pltpu.VMEM((RPT,), jnp.float32),
           pltpu.VMEM_SHARED((NPAD,), jnp.float32)]
        + [pltpu.SemaphoreType.DMA for _ in range(2 * RI)]
    ),
    compiler_params=pltpu.CompilerParams(
        needs_layout_passes=False, disable_bounds_checks=True),
)
def _deg_kernel(col_hbm, w_hbm, deg_out, *sc):
    cbuf, wbuf = list(sc[0:RI]), list(sc[RI:2 * RI])
    zbuf, deg_sh = sc[2 * RI], sc[2 * RI + 1]
    sem_i = list(sc[2 * RI + 2:2 * RI + 2 + RI])
    sem_s = list(sc[2 * RI + 2 + RI:])
    c = lax.axis_index("c")
    s = lax.axis_index("s")
    base = _wid() * EPT

    def issue_idx(t, ch):
        off = base + ch * CH
        pltpu.async_copy(col_hbm.at[pl.ds(off, CH)], cbuf[t], sem_i[t])
        pltpu.async_copy(w_hbm.at[pl.ds(off, CH)], wbuf[t], sem_i[t])

    def wait_idx(t, ch):
        off = base + ch * CH
        pltpu.make_async_copy(
            col_hbm.at[pl.ds(off, CH)], cbuf[t], sem_i[t]).wait()
        pltpu.make_async_copy(
            w_hbm.at[pl.ds(off, CH)], wbuf[t], sem_i[t]).wait()

    issue_idx(0, 0)
    issue_idx(1, 1)
    _zero16(zbuf, RPT // 16)
    pltpu.sync_copy(zbuf, deg_sh.at[pl.ds(s * RPT, RPT)])
    plsc.subcore_barrier()

    def group(h, _):
        for u in range(4):
            j = h * 4 + u
            t = u % 4

            @pl.when((j >= 2) & (j <= NCHUNK + 1))
            def _():
                t2 = (u - 2) % 4
                pltpu.make_async_copy(
                    wbuf[t2], deg_sh.at[cbuf[t2]], sem_s[t2]).wait()

            @pl.when(j <= NCHUNK - 3)
            def _():
                issue_idx((u + 2) % 4, j + 2)

            @pl.when(j <= NCHUNK - 1)
            def _():
                wait_idx(t, j)
                pltpu.async_copy(wbuf[t], deg_sh.at[cbuf[t]], sem_s[t],
                                 add=True)
        return 0

    lax.fori_loop(0, 32, group, 0)
    plsc.subcore_barrier()
    pltpu.sync_copy(deg_sh.at[pl.ds(s * RPT, RPT)],
                    deg_out.at[c, pl.ds(s * RPT, RPT)])


def _make_prop_kernel(nfeat, with_norm):
    outs = [jax.ShapeDtypeStruct((2, NPAD, nfeat), jnp.float32)]
    if with_norm:
        outs = [jax.ShapeDtypeStruct((E,), jnp.float32)] + outs
    ZR = 40
    scratch = (
        [pltpu.VMEM((CH,), jnp.int32) for _ in range(RI)]
        + [pltpu.VMEM((CH,), jnp.int32) for _ in range(RI)]
        + [pltpu.VMEM((CH,), jnp.float32) for _ in range(RI)]
        + [pltpu.VMEM((CH, nfeat), jnp.float32) for _ in range(RR)]
        + [pltpu.VMEM((ZR, nfeat), jnp.float32),
           pltpu.VMEM_SHARED((NPAD, nfeat), jnp.float32)]
        + [pltpu.SemaphoreType.DMA for _ in range(RI + RR + RR)]
    )
    if with_norm:
        scratch = (
            [pltpu.VMEM((CH,), jnp.float32) for _ in range(RI)]
            + [pltpu.VMEM((NPAD,), jnp.float32)]
            + scratch
            + [pltpu.SemaphoreType.DMA for _ in range(RI)]
        )

    def body(*refs):
        if with_norm:
            (row_hbm, col_hbm, w_hbm, dinv_hbm, xw_hbm,
             norm_out, part_out) = refs[:7]
            wbuf = list(refs[7:7 + RI])
            dinv_v = refs[7 + RI]
            rest = refs[8 + RI:]
        else:
            (row_hbm, col_hbm, norm_hbm, xw_hbm, part_out) = refs[:5]
            rest = refs[5:]
        ridx = list(rest[0:RI])
        cidx = list(rest[RI:2 * RI])
        nbuf = list(rest[2 * RI:3 * RI])
        rows = list(rest[3 * RI:3 * RI + RR])
        zbuf = rest[3 * RI + RR]
        out_sh = rest[3 * RI + RR + 1]
        sems = rest[3 * RI + RR + 2:]
        sem_i = list(sems[0:RI])
        sem_g = list(sems[RI:RI + RR])
        sem_s = list(sems[RI + RR:RI + RR + RR])
        if with_norm:
            sem_n = list(sems[RI + RR + RR:])

        c = lax.axis_index("c")
        s = lax.axis_index("s")
        base = _wid() * EPT

        def issue_idx(t, ch):
            off = base + ch * CH
            pltpu.async_copy(row_hbm.at[pl.ds(off, CH)], ridx[t], sem_i[t])
            pltpu.async_copy(col_hbm.at[pl.ds(off, CH)], cidx[t], sem_i[t])
            if with_norm:
                pltpu.async_copy(w_hbm.at[pl.ds(off, CH)], wbuf[t], sem_i[t])
            else:
                pltpu.async_copy(norm_hbm.at[pl.ds(off, CH)], nbuf[t],
                                 sem_i[t])

        def wait_idx(t, ch):
            off = base + ch * CH
            pltpu.make_async_copy(
                row_hbm.at[pl.ds(off, CH)], ridx[t], sem_i[t]).wait()
            pltpu.make_async_copy(
                col_hbm.at[pl.ds(off, CH)], cidx[t], sem_i[t]).wait()
            if with_norm:
                pltpu.make_async_copy(
                    w_hbm.at[pl.ds(off, CH)], wbuf[t], sem_i[t]).wait()
            else:
                pltpu.make_async_copy(
                    norm_hbm.at[pl.ds(off, CH)], nbuf[t], sem_i[t]).wait()

        issue_idx(0, 0)
        issue_idx(1, 1)
        if with_norm:
            pltpu.async_copy(dinv_hbm, dinv_v, sem_g[1])

        def zz(i, _):
            for f in range(nfeat // 16):
                zbuf[i, pl.ds(f * 16, 16)] = jnp.zeros((16,), jnp.float32)
            return 0

        lax.fori_loop(0, ZR, zz, 0)
        nz = RPT // ZR
        for q in range(nz):
            pltpu.async_copy(zbuf, out_sh.at[pl.ds(s * RPT + q * ZR, ZR)],
                             sem_s[q % RR])
        wait_idx(0, 0)
        if with_norm:
            pltpu.make_async_copy(dinv_hbm, dinv_v, sem_g[1]).wait()
        pltpu.async_copy(xw_hbm.at[ridx[0]], rows[0], sem_g[0])
        for q in range(nz):
            pltpu.make_async_copy(
                zbuf, out_sh.at[pl.ds(s * RPT + q * ZR, ZR)],
                sem_s[q % RR]).wait()
        plsc.subcore_barrier()

        def group(h, _):
            for u in range(UNROLL):
                j = h * UNROLL + u
                ti = u % RI
                tr = u % RR

                @pl.when((j >= 2) & (j <= NCHUNK + 1))
                def _():
                    pltpu.make_async_copy(
                        rows[(u - 2) % RR],
                        out_sh.at[cidx[(u - 2) % RI]],
                        sem_s[(u - 2) % RR]).wait()

                @pl.when(j <= NCHUNK - 3)
                def _():
                    issue_idx((u + 2) % RI, j + 2)

                @pl.when((j >= 0) & (j <= NCHUNK - 2))
                def _():
                    wait_idx((u + 1) % RI, j + 1)
                    pltpu.async_copy(xw_hbm.at[ridx[(u + 1) % RI]],
                                     rows[(u + 1) % RR],
                                     sem_g[(u + 1) % RR])

                if with_norm:
                    @pl.when((j >= 4) & (j <= NCHUNK + 3))
                    def _():
                        off = base + (j - 4) * CH
                        pltpu.make_async_copy(
                            nbuf[ti], norm_out.at[pl.ds(off, CH)],
                            sem_n[ti]).wait()

                    @pl.when(j <= NCHUNK - 1)
                    def _():
                        for g in range(CH // 16):
                            sl = pl.ds(g * 16, 16)
                            dr = plsc.load_gather(dinv_v, [ridx[ti][sl]])
                            dc = plsc.load_gather(dinv_v, [cidx[ti][sl]])
                            nbuf[ti][sl] = dr * wbuf[ti][sl] * dc
                        pltpu.async_copy(
                            nbuf[ti],
                            norm_out.at[pl.ds(base + j * CH, CH)],
                            sem_n[ti])

                @pl.when(j <= NCHUNK - 1)
                def _():
                    pltpu.make_async_copy(
                        xw_hbm.at[ridx[ti]], rows[tr], sem_g[tr]).wait()

                    def sc(k, _):
                        bc = plsc.load_gather(
                            nbuf[ti], [jnp.full((16,), k, jnp.int32)])
                        for f in range(nfeat // 16):
                            sl = pl.ds(f * 16, 16)
                            rows[tr][k, sl] = rows[tr][k, sl] * bc
                        return 0

                    lax.fori_loop(0, CH, sc, 0)
                    pltpu.async_copy(rows[tr], out_sh.at[cidx[ti]],
                                     sem_s[tr], add=True)
            return 0

        lax.fori_loop(0, (NCHUNK + 7 + UNROLL - 1) // UNROLL, group, 0)
        plsc.subcore_barrier()
        pltpu.sync_copy(out_sh.at[pl.ds(s * RPT, RPT)],
                        part_out.at[c, pl.ds(s * RPT, RPT)])

    return pl.kernel(
        body,
        out_type=outs if with_norm else outs[0],
        mesh=plsc.VectorSubcoreMesh(**_MESH),
        scratch_types=scratch,
        compiler_params=pltpu.CompilerParams(
            needs_layout_passes=False,
            disable_bounds_checks=True,
            use_tc_tiling_on_sc=None if nfeat % 128 == 0 else False,
        ),
    )


_prop1 = _make_prop_kernel(F1, with_norm=True)
_prop2 = _make_prop_kernel(F2, with_norm=False)


def _matmul1_body(x_ref, w_ref, deg_ref, o_ref, dinv_ref, dinv2_ref):
    o_ref[...] = lax.dot_general(
        x_ref[...], w_ref[...], (((1,), (1,)), ((), ())),
        preferred_element_type=jnp.float32)
    deg = deg_ref[0, :] + deg_ref[1, :] + 1.0
    dinv = jnp.where(deg > 0, lax.rsqrt(deg), 0.0)
    dinv_ref[...] = dinv
    dinv2_ref[...] = dinv * dinv


def _mid_body(p0_ref, p1_ref, xw_ref, dinv2_ref, b_ref, w2_ref, o_ref):
    h = (p0_ref[0] + p1_ref[0] + dinv2_ref[...] * xw_ref[...] + b_ref[...])
    h = jnp.maximum(h, 0.0)
    o_ref[...] = lax.dot_general(
        h, w2_ref[...], (((1,), (1,)), ((), ())),
        preferred_element_type=jnp.float32)


def _final_body(p0_ref, p1_ref, xw_ref, dinv2_ref, b_ref, o_ref):
    t = (p0_ref[0] + p1_ref[0] + dinv2_ref[...] * xw_ref[...] + b_ref[...])
    m = jnp.max(t, axis=1, keepdims=True)
    lse = jnp.log(jnp.sum(jnp.exp(t - m), axis=1, keepdims=True)) + m
    o_ref[...] = t - lse


_ROWB = 1000


def kernel(x, edge_index, edge_weight, W1, b1, W2, b2):
    row = edge_index[0].astype(jnp.int32)
    col = edge_index[1].astype(jnp.int32)
    w = edge_weight.astype(jnp.float32)

    deg_p = _deg_kernel(col, w)

    _DB = NPAD // (N // _ROWB)
    xw1, dinv, dinv2 = pl.pallas_call(
        _matmul1_body,
        grid=(N // _ROWB,),
        in_specs=[pl.BlockSpec((_ROWB, F1), lambda i: (i, 0)),
                  pl.BlockSpec((F1, F1), lambda i: (0, 0)),
                  pl.BlockSpec((2, _DB), lambda i: (0, i))],
        out_specs=[pl.BlockSpec((_ROWB, F1), lambda i: (i, 0)),
                   pl.BlockSpec((_DB,), lambda i: (i,)),
                   pl.BlockSpec((_DB,), lambda i: (i,))],
        out_shape=[jax.ShapeDtypeStruct((N, F1), jnp.float32),
                   jax.ShapeDtypeStruct((NPAD,), jnp.float32),
                   jax.ShapeDtypeStruct((NPAD,), jnp.float32)],
    )(x, W1, deg_p)

    norm, part1 = _prop1(row, col, w, dinv, xw1)

    dinv2c = dinv2[:N, None]
    xw2 = pl.pallas_call(
        _mid_body,
        grid=(N // _ROWB,),
        in_specs=[pl.BlockSpec((1, _ROWB, F1), lambda i: (0, i, 0)),
                  pl.BlockSpec((1, _ROWB, F1), lambda i: (1, i, 0)),
                  pl.BlockSpec((_ROWB, F1), lambda i: (i, 0)),
                  pl.BlockSpec((_ROWB, 1), lambda i: (i, 0)),
                  pl.BlockSpec((1, F1), lambda i: (0, 0)),
                  pl.BlockSpec((F2, F1), lambda i: (0, 0))],
        out_specs=pl.BlockSpec((_ROWB, F2), lambda i: (i, 0)),
        out_shape=jax.ShapeDtypeStruct((N, F2), jnp.float32),
    )(part1, part1, xw1, dinv2c, b1[None, :], W2)

    part2 = _prop2(row, col, norm, xw2)

    out = pl.pallas_call(
        _final_body,
        grid=(N // _ROWB,),
        in_specs=[pl.BlockSpec((1, _ROWB, F2), lambda i: (0, i, 0)),
                  pl.BlockSpec((1, _ROWB, F2), lambda i: (1, i, 0)),
                  pl.BlockSpec((_ROWB, F2), lambda i: (i, 0)),
                  pl.BlockSpec((_ROWB, 1), lambda i: (i, 0)),
                  pl.BlockSpec((1, F2), lambda i: (0, 0))],
        out_specs=pl.BlockSpec((_ROWB, F2), lambda i: (i, 0)),
        out_shape=jax.ShapeDtypeStruct((N, F2), jnp.float32),
    )(part2, part2, xw2, dinv2c, b2[None, :])

    return out


# speedup vs baseline: 1.0681x; 1.0151x over previous
import functools

import jax
import jax.numpy as jnp
from jax import lax
from jax.experimental import pallas as pl
from jax.experimental.pallas import tpu as pltpu
from jax.experimental.pallas import tpu_sc as plsc

N = 10000
NPAD = 10240
E = 320000
F1 = 128
F2 = 64
NW = 32
EPT = E // NW
CH = 80
NCHUNK = EPT // CH
RPT = NPAD // 16

RI = 4
RR = 3
UNROLL = 12

_MESH = dict(core_axis_name="c", subcore_axis_name="s")


def _wid():
    return lax.axis_index("s") * 2 + lax.axis_index("c")


def _zero16(buf, nwords):
    def zb(i, _):
        buf[pl.ds(i * 16, 16)] = jnp.zeros((16,), jnp.float32)
        return 0
    lax.fori_loop(0, nwords, zb, 0)


@functools.partial(
    pl.kernel,
    out_type=jax.ShapeDtypeStruct((2, NPAD), jnp.float32),
    mesh=plsc.VectorSubcoreMesh(**_MESH),
    scratch_types=(
        [pltpu.VMEM((CH,), jnp.int32) for _ in range(RI)]
        + [pltpu.VMEM((CH,), jnp.float32) for _ in range(RI)]
        + [pltpu.VMEM((RPT,), jnp.float32),
           pltpu.VMEM_SHARED((NPAD,), jnp.float32)]
        + [pltpu.SemaphoreType.DMA for _ in range(2 * RI)]
    ),
    compiler_params=pltpu.CompilerParams(
        needs_layout_passes=False, disable_bounds_checks=True),
)
def _deg_kernel(col_hbm, w_hbm, deg_out, *sc):
    cbuf, wbuf = list(sc[0:RI]), list(sc[RI:2 * RI])
    zbuf, deg_sh = sc[2 * RI], sc[2 * RI + 1]
    sem_i = list(sc[2 * RI + 2:2 * RI + 2 + RI])
    sem_s = list(sc[2 * RI + 2 + RI:])
    c = lax.axis_index("c")
    s = lax.axis_index("s")
    base = _wid() * EPT

    def issue_idx(t, ch):
        off = base + ch * CH
        pltpu.async_copy(col_hbm.at[pl.ds(off, CH)], cbuf[t], sem_i[t])
        pltpu.async_copy(w_hbm.at[pl.ds(off, CH)], wbuf[t], sem_i[t])

    def wait_idx(t, ch):
        off = base + ch * CH
        pltpu.make_async_copy(
            col_hbm.at[pl.ds(off, CH)], cbuf[t], sem_i[t]).wait()
        pltpu.make_async_copy(
            w_hbm.at[pl.ds(off, CH)], wbuf[t], sem_i[t]).wait()

    issue_idx(0, 0)
    issue_idx(1, 1)
    _zero16(zbuf, RPT // 16)
    pltpu.sync_copy(zbuf, deg_sh.at[pl.ds(s * RPT, RPT)])
    plsc.subcore_barrier()

    def group(h, _):
        for u in range(4):
            j = h * 4 + u
            t = u % 4

            @pl.when((j >= 2) & (j <= NCHUNK + 1))
            def _():
                t2 = (u - 2) % 4
                pltpu.make_async_copy(
                    wbuf[t2], deg_sh.at[cbuf[t2]], sem_s[t2]).wait()

            @pl.when(j <= NCHUNK - 3)
            def _():
                issue_idx((u + 2) % 4, j + 2)

            @pl.when(j <= NCHUNK - 1)
            def _():
                wait_idx(t, j)
                pltpu.async_copy(wbuf[t], deg_sh.at[cbuf[t]], sem_s[t],
                                 add=True)
        return 0

    lax.fori_loop(0, 32, group, 0)
    plsc.subcore_barrier()
    pltpu.sync_copy(deg_sh.at[pl.ds(s * RPT, RPT)],
                    deg_out.at[c, pl.ds(s * RPT, RPT)])


def _make_prop_kernel(nfeat, with_norm):
    outs = [jax.ShapeDtypeStruct((2, NPAD, nfeat), jnp.float32)]
    if with_norm:
        outs = [jax.ShapeDtypeStruct((E,), jnp.float32)] + outs
    ZR = 40
    scratch = (
        [pltpu.VMEM((CH,), jnp.int32) for _ in range(RI)]
        + [pltpu.VMEM((CH,), jnp.int32) for _ in range(RI)]
        + [pltpu.VMEM((CH,), jnp.float32) for _ in range(RI)]
        + [pltpu.VMEM((CH, nfeat), jnp.float32) for _ in range(RR)]
        + [pltpu.VMEM((ZR, nfeat), jnp.float32),
           pltpu.VMEM_SHARED((NPAD, nfeat), jnp.float32)]
        + [pltpu.SemaphoreType.DMA for _ in range(RI + RR + RR)]
    )
    if with_norm:
        scratch = (
            [pltpu.VMEM((CH,), jnp.float32) for _ in range(RI)]
            + [pltpu.VMEM((NPAD,), jnp.float32)]
            + scratch
            + [pltpu.SemaphoreType.DMA for _ in range(RI)]
        )

    def body(*refs):
        if with_norm:
            (row_hbm, col_hbm, w_hbm, dinv_hbm, xw_hbm,
             norm_out, part_out) = refs[:7]
            wbuf = list(refs[7:7 + RI])
            dinv_v = refs[7 + RI]
            rest = refs[8 + RI:]
        else:
            (row_hbm, col_hbm, norm_hbm, xw_hbm, part_out) = refs[:5]
            rest = refs[5:]
        ridx = list(rest[0:RI])
        cidx = list(rest[RI:2 * RI])
        nbuf = list(rest[2 * RI:3 * RI])
        rows = list(rest[3 * RI:3 * RI + RR])
        zbuf = rest[3 * RI + RR]
        out_sh = rest[3 * RI + RR + 1]
        sems = rest[3 * RI + RR + 2:]
        sem_i = list(sems[0:RI])
        sem_g = list(sems[RI:RI + RR])
        sem_s = list(sems[RI + RR:RI + RR + RR])
        if with_norm:
            sem_n = list(sems[RI + RR + RR:])

        c = lax.axis_index("c")
        s = lax.axis_index("s")
        base = _wid() * EPT

        def issue_idx(t, ch):
            off = base + ch * CH
            pltpu.async_copy(row_hbm.at[pl.ds(off, CH)], ridx[t], sem_i[t])
            pltpu.async_copy(col_hbm.at[pl.ds(off, CH)], cidx[t], sem_i[t])
            if with_norm:
                pltpu.async_copy(w_hbm.at[pl.ds(off, CH)], wbuf[t], sem_i[t])
            else:
                pltpu.async_copy(norm_hbm.at[pl.ds(off, CH)], nbuf[t],
                                 sem_i[t])

        def wait_idx(t, ch):
            off = base + ch * CH
            pltpu.make_async_copy(
                row_hbm.at[pl.ds(off, CH)], ridx[t], sem_i[t]).wait()
            pltpu.make_async_copy(
                col_hbm.at[pl.ds(off, CH)], cidx[t], sem_i[t]).wait()
            if with_norm:
                pltpu.make_async_copy(
                    w_hbm.at[pl.ds(off, CH)], wbuf[t], sem_i[t]).wait()
            else:
                pltpu.make_async_copy(
                    norm_hbm.at[pl.ds(off, CH)], nbuf[t], sem_i[t]).wait()

        issue_idx(0, 0)
        issue_idx(1, 1)
        if with_norm:
            pltpu.async_copy(dinv_hbm, dinv_v, sem_g[1])

        def zz(i, _):
            for f in range(nfeat // 16):
                zbuf[i, pl.ds(f * 16, 16)] = jnp.zeros((16,), jnp.float32)
            return 0

        lax.fori_loop(0, ZR, zz, 0)
        nz = RPT // ZR
        for q in range(nz):
            pltpu.async_copy(zbuf, out_sh.at[pl.ds(s * RPT + q * ZR, ZR)],
                             sem_s[q % RR])
        wait_idx(0, 0)
        if with_norm:
            pltpu.make_async_copy(dinv_hbm, dinv_v, sem_g[1]).wait()
        pltpu.async_copy(xw_hbm.at[ridx[0]], rows[0], sem_g[0])
        for q in range(nz):
            pltpu.make_async_copy(
                zbuf, out_sh.at[pl.ds(s * RPT + q * ZR, ZR)],
                sem_s[q % RR]).wait()
        plsc.subcore_barrier()

        def group(h, _):
            for u in range(UNROLL):
                j = h * UNROLL + u
                ti = u % RI
                tr = u % RR

                @pl.when((j >= 2) & (j <= NCHUNK + 1))
                def _():
                    pltpu.make_async_copy(
                        rows[(u - 2) % RR],
                        out_sh.at[cidx[(u - 2) % RI]],
                        sem_s[(u - 2) % RR]).wait()

                @pl.when(j <= NCHUNK - 3)
                def _():
                    issue_idx((u + 2) % RI, j + 2)

                @pl.when((j >= 0) & (j <= NCHUNK - 2))
                def _():
                    wait_idx((u + 1) % RI, j + 1)
                    pltpu.async_copy(xw_hbm.at[ridx[(u + 1) % RI]],
                                     rows[(u + 1) % RR],
                                     sem_g[(u + 1) % RR])

                if with_norm:
                    @pl.when((j >= 4) & (j <= NCHUNK + 3))
                    def _():
                        off = base + (j - 4) * CH
                        pltpu.make_async_copy(
                            nbuf[ti], norm_out.at[pl.ds(off, CH)],
                            sem_n[ti]).wait()

                    @pl.when(j <= NCHUNK - 1)
                    def _():
                        for g in range(CH // 16):
                            sl = pl.ds(g * 16, 16)
                            dr = plsc.load_gather(dinv_v, [ridx[ti][sl]])
                            dc = plsc.load_gather(dinv_v, [cidx[ti][sl]])
                            nbuf[ti][sl] = dr * wbuf[ti][sl] * dc
                        pltpu.async_copy(
                            nbuf[ti],
                            norm_out.at[pl.ds(base + j * CH, CH)],
                            sem_n[ti])

                @pl.when(j <= NCHUNK - 1)
                def _():
                    pltpu.make_async_copy(
                        xw_hbm.at[ridx[ti]], rows[tr], sem_g[tr]).wait()

                    def sc(k, _):
                        bc = plsc.load_gather(
                            nbuf[ti], [jnp.full((16,), k, jnp.int32)])
                        for f in range(nfeat // 16):
                            sl = pl.ds(f * 16, 16)
                            rows[tr][k, sl] = rows[tr][k, sl] * bc
                        return 0

                    lax.fori_loop(0, CH, sc, 0)
                    pltpu.async_copy(rows[tr], out_sh.at[cidx[ti]],
                                     sem_s[tr], add=True)
            return 0

        lax.fori_loop(0, (NCHUNK + 7 + UNROLL - 1) // UNROLL, group, 0)
        plsc.subcore_barrier()
        pltpu.sync_copy(out_sh.at[pl.ds(s * RPT, RPT)],
                        part_out.at[c, pl.ds(s * RPT, RPT)])

    return pl.kernel(
        body,
        out_type=outs if with_norm else outs[0],
        mesh=plsc.VectorSubcoreMesh(**_MESH),
        scratch_types=scratch,
        compiler_params=pltpu.CompilerParams(
            needs_layout_passes=False,
            disable_bounds_checks=True,
            use_tc_tiling_on_sc=None if nfeat % 128 == 0 else False,
        ),
    )


_prop1 = _make_prop_kernel(F1, with_norm=True)
_prop2 = _make_prop_kernel(F2, with_norm=False)


def _matmul1_body(x_ref, w_ref, o_ref):
    o_ref[...] = lax.dot_general(
        x_ref[...], w_ref[...], (((1,), (1,)), ((), ())),
        preferred_element_type=jnp.float32)


def _dinv_body(deg_ref, dinv_ref, dinv2_ref):
    deg = deg_ref[0, :] + deg_ref[1, :] + 1.0
    dinv = jnp.where(deg > 0, lax.rsqrt(deg), 0.0)
    dinv_ref[...] = dinv
    dinv2_ref[...] = dinv * dinv


def _mid_body(p0_ref, p1_ref, xw_ref, dinv2_ref, b_ref, w2_ref, o_ref):
    h = (p0_ref[0] + p1_ref[0] + dinv2_ref[...] * xw_ref[...] + b_ref[...])
    h = jnp.maximum(h, 0.0)
    o_ref[...] = lax.dot_general(
        h, w2_ref[...], (((1,), (1,)), ((), ())),
        preferred_element_type=jnp.float32)


def _final_body(p0_ref, p1_ref, xw_ref, dinv2_ref, b_ref, o_ref):
    t = (p0_ref[0] + p1_ref[0] + dinv2_ref[...] * xw_ref[...] + b_ref[...])
    m = jnp.max(t, axis=1, keepdims=True)
    lse = jnp.log(jnp.sum(jnp.exp(t - m), axis=1, keepdims=True)) + m
    o_ref[...] = t - lse


_ROWB = 1000


def kernel(x, edge_index, edge_weight, W1, b1, W2, b2):
    row = edge_index[0].astype(jnp.int32)
    col = edge_index[1].astype(jnp.int32)
    w = edge_weight.astype(jnp.float32)

    deg_p = _deg_kernel(col, w)

    xw1 = pl.pallas_call(
        _matmul1_body,
        grid=(N // _ROWB,),
        in_specs=[pl.BlockSpec((_ROWB, F1), lambda i: (i, 0)),
                  pl.BlockSpec((F1, F1), lambda i: (0, 0))],
        out_specs=pl.BlockSpec((_ROWB, F1), lambda i: (i, 0)),
        out_shape=jax.ShapeDtypeStruct((N, F1), jnp.float32),
    )(x, W1)

    dinv, dinv2 = pl.pallas_call(
        _dinv_body,
        out_shape=[jax.ShapeDtypeStruct((NPAD,), jnp.float32),
                   jax.ShapeDtypeStruct((NPAD,), jnp.float32)],
    )(deg_p)

    norm, part1 = _prop1(row, col, w, dinv, xw1)

    dinv2c = dinv2[:N, None]
    xw2 = pl.pallas_call(
        _mid_body,
        grid=(N // _ROWB,),
        in_specs=[pl.BlockSpec((1, _ROWB, F1), lambda i: (0, i, 0)),
                  pl.BlockSpec((1, _ROWB, F1), lambda i: (1, i, 0)),
                  pl.BlockSpec((_ROWB, F1), lambda i: (i, 0)),
                  pl.BlockSpec((_ROWB, 1), lambda i: (i, 0)),
                  pl.BlockSpec((1, F1), lambda i: (0, 0)),
                  pl.BlockSpec((F2, F1), lambda i: (0, 0))],
        out_specs=pl.BlockSpec((_ROWB, F2), lambda i: (i, 0)),
        out_shape=jax.ShapeDtypeStruct((N, F2), jnp.float32),
    )(part1, part1, xw1, dinv2c, b1[None, :], W2)

    part2 = _prop2(row, col, norm, xw2)

    out = pl.pallas_call(
        _final_body,
        grid=(N // _ROWB,),
        in_specs=[pl.BlockSpec((1, _ROWB, F2), lambda i: (0, i, 0)),
                  pl.BlockSpec((1, _ROWB, F2), lambda i: (1, i, 0)),
                  pl.BlockSpec((_ROWB, F2), lambda i: (i, 0)),
                  pl.BlockSpec((_ROWB, 1), lambda i: (i, 0)),
                  pl.BlockSpec((1, F2), lambda i: (0, 0))],
        out_specs=pl.BlockSpec((_ROWB, F2), lambda i: (i, 0)),
        out_shape=jax.ShapeDtypeStruct((N, F2), jnp.float32),
    )(part2, part2, xw2, dinv2c, b2[None, :])

    return out


# speedup vs baseline: 1.1083x; 1.0376x over previous
import functools

import jax
import jax.numpy as jnp
from jax import lax
from jax.experimental import pallas as pl
from jax.experimental.pallas import tpu as pltpu
from jax.experimental.pallas import tpu_sc as plsc

N = 10000
NPAD = 10240
E = 320000
F1 = 128
F2 = 64
NW = 32
EPT = E // NW
CH = 80
NCHUNK = EPT // CH
RPT = NPAD // 16

RI = 4
RR = 3
UNROLL = 12

_MESH = dict(core_axis_name="c", subcore_axis_name="s")


def _wid():
    return lax.axis_index("s") * 2 + lax.axis_index("c")


def _zero16(buf, nwords):
    def zb(i, _):
        buf[pl.ds(i * 16, 16)] = jnp.zeros((16,), jnp.float32)
        return 0
    lax.fori_loop(0, nwords, zb, 0)


@functools.partial(
    pl.kernel,
    out_type=jax.ShapeDtypeStruct((2, NPAD), jnp.float32),
    mesh=plsc.VectorSubcoreMesh(**_MESH),
    scratch_types=(
        [pltpu.VMEM((CH,), jnp.int32) for _ in range(RI)]
        + [pltpu.VMEM((CH,), jnp.float32) for _ in range(RI)]
        + [pltpu.VMEM((RPT,), jnp.float32),
           pltpu.VMEM_SHARED((NPAD,), jnp.float32)]
        + [pltpu.SemaphoreType.DMA for _ in range(2 * RI)]
    ),
    compiler_params=pltpu.CompilerParams(
        needs_layout_passes=False, disable_bounds_checks=True),
)
def _deg_kernel(col_hbm, w_hbm, deg_out, *sc):
    cbuf, wbuf = list(sc[0:RI]), list(sc[RI:2 * RI])
    zbuf, deg_sh = sc[2 * RI], sc[2 * RI + 1]
    sem_i = list(sc[2 * RI + 2:2 * RI + 2 + RI])
    sem_s = list(sc[2 * RI + 2 + RI:])
    c = lax.axis_index("c")
    s = lax.axis_index("s")
    base = _wid() * EPT

    def issue_idx(t, ch):
        off = base + ch * CH
        pltpu.async_copy(col_hbm.at[pl.ds(off, CH)], cbuf[t], sem_i[t])
        pltpu.async_copy(w_hbm.at[pl.ds(off, CH)], wbuf[t], sem_i[t])

    def wait_idx(t, ch):
        off = base + ch * CH
        pltpu.make_async_copy(
            col_hbm.at[pl.ds(off, CH)], cbuf[t], sem_i[t]).wait()
        pltpu.make_async_copy(
            w_hbm.at[pl.ds(off, CH)], wbuf[t], sem_i[t]).wait()

    issue_idx(0, 0)
    issue_idx(1, 1)
    _zero16(zbuf, RPT // 16)
    pltpu.sync_copy(zbuf, deg_sh.at[pl.ds(s * RPT, RPT)])
    plsc.subcore_barrier()

    def group(h, _):
        for u in range(4):
            j = h * 4 + u
            t = u % 4

            @pl.when((j >= 2) & (j <= NCHUNK + 1))
            def _():
                t2 = (u - 2) % 4
                pltpu.make_async_copy(
                    wbuf[t2], deg_sh.at[cbuf[t2]], sem_s[t2]).wait()

            @pl.when(j <= NCHUNK - 3)
            def _():
                issue_idx((u + 2) % 4, j + 2)

            @pl.when(j <= NCHUNK - 1)
            def _():
                wait_idx(t, j)
                pltpu.async_copy(wbuf[t], deg_sh.at[cbuf[t]], sem_s[t],
                                 add=True)
        return 0

    lax.fori_loop(0, 32, group, 0)
    plsc.subcore_barrier()
    pltpu.sync_copy(deg_sh.at[pl.ds(s * RPT, RPT)],
                    deg_out.at[c, pl.ds(s * RPT, RPT)])


def _make_prop_kernel(nfeat, with_norm, chunk=CH):
    CH = chunk
    NCHUNK = EPT // chunk
    outs = [jax.ShapeDtypeStruct((2, NPAD, nfeat), jnp.float32)]
    if with_norm:
        outs = [jax.ShapeDtypeStruct((E,), jnp.float32)] + outs
    ZR = 40
    scratch = (
        [pltpu.VMEM((CH,), jnp.int32) for _ in range(RI)]
        + [pltpu.VMEM((CH,), jnp.int32) for _ in range(RI)]
        + [pltpu.VMEM((CH,), jnp.float32) for _ in range(RI)]
        + [pltpu.VMEM((CH, nfeat), jnp.float32) for _ in range(RR)]
        + [pltpu.VMEM((ZR, nfeat), jnp.float32),
           pltpu.VMEM_SHARED((NPAD, nfeat), jnp.float32)]
        + [pltpu.SemaphoreType.DMA for _ in range(RI + RR + RR)]
    )
    if with_norm:
        scratch = (
            [pltpu.VMEM((CH,), jnp.float32) for _ in range(RI)]
            + [pltpu.VMEM((NPAD,), jnp.float32)]
            + scratch
            + [pltpu.SemaphoreType.DMA for _ in range(RI)]
        )

    def body(*refs):
        if with_norm:
            (row_hbm, col_hbm, w_hbm, dinv_hbm, xw_hbm,
             norm_out, part_out) = refs[:7]
            wbuf = list(refs[7:7 + RI])
            dinv_v = refs[7 + RI]
            rest = refs[8 + RI:]
        else:
            (row_hbm, col_hbm, norm_hbm, xw_hbm, part_out) = refs[:5]
            rest = refs[5:]
        ridx = list(rest[0:RI])
        cidx = list(rest[RI:2 * RI])
        nbuf = list(rest[2 * RI:3 * RI])
        rows = list(rest[3 * RI:3 * RI + RR])
        zbuf = rest[3 * RI + RR]
        out_sh = rest[3 * RI + RR + 1]
        sems = rest[3 * RI + RR + 2:]
        sem_i = list(sems[0:RI])
        sem_g = list(sems[RI:RI + RR])
        sem_s = list(sems[RI + RR:RI + RR + RR])
        if with_norm:
            sem_n = list(sems[RI + RR + RR:])

        c = lax.axis_index("c")
        s = lax.axis_index("s")
        base = _wid() * EPT

        def issue_idx(t, ch):
            off = base + ch * CH
            pltpu.async_copy(row_hbm.at[pl.ds(off, CH)], ridx[t], sem_i[t])
            pltpu.async_copy(col_hbm.at[pl.ds(off, CH)], cidx[t], sem_i[t])
            if with_norm:
                pltpu.async_copy(w_hbm.at[pl.ds(off, CH)], wbuf[t], sem_i[t])
            else:
                pltpu.async_copy(norm_hbm.at[pl.ds(off, CH)], nbuf[t],
                                 sem_i[t])

        def wait_idx(t, ch):
            off = base + ch * CH
            pltpu.make_async_copy(
                row_hbm.at[pl.ds(off, CH)], ridx[t], sem_i[t]).wait()
            pltpu.make_async_copy(
                col_hbm.at[pl.ds(off, CH)], cidx[t], sem_i[t]).wait()
            if with_norm:
                pltpu.make_async_copy(
                    w_hbm.at[pl.ds(off, CH)], wbuf[t], sem_i[t]).wait()
            else:
                pltpu.make_async_copy(
                    norm_hbm.at[pl.ds(off, CH)], nbuf[t], sem_i[t]).wait()

        issue_idx(0, 0)
        issue_idx(1, 1)
        if with_norm:
            pltpu.async_copy(dinv_hbm, dinv_v, sem_g[1])

        def zz(i, _):
            for f in range(nfeat // 16):
                zbuf[i, pl.ds(f * 16, 16)] = jnp.zeros((16,), jnp.float32)
            return 0

        lax.fori_loop(0, ZR, zz, 0)
        nz = RPT // ZR
        for q in range(nz):
            pltpu.async_copy(zbuf, out_sh.at[pl.ds(s * RPT + q * ZR, ZR)],
                             sem_s[q % RR])
        wait_idx(0, 0)
        if with_norm:
            pltpu.make_async_copy(dinv_hbm, dinv_v, sem_g[1]).wait()
        pltpu.async_copy(xw_hbm.at[ridx[0]], rows[0], sem_g[0])
        for q in range(nz):
            pltpu.make_async_copy(
                zbuf, out_sh.at[pl.ds(s * RPT + q * ZR, ZR)],
                sem_s[q % RR]).wait()
        plsc.subcore_barrier()

        def group(h, _):
            for u in range(UNROLL):
                j = h * UNROLL + u
                ti = u % RI
                tr = u % RR

                @pl.when((j >= 2) & (j <= NCHUNK + 1))
                def _():
                    pltpu.make_async_copy(
                        rows[(u - 2) % RR],
                        out_sh.at[cidx[(u - 2) % RI]],
                        sem_s[(u - 2) % RR]).wait()

                @pl.when(j <= NCHUNK - 3)
                def _():
                    issue_idx((u + 2) % RI, j + 2)

                @pl.when((j >= 0) & (j <= NCHUNK - 2))
                def _():
                    wait_idx((u + 1) % RI, j + 1)
                    pltpu.async_copy(xw_hbm.at[ridx[(u + 1) % RI]],
                                     rows[(u + 1) % RR],
                                     sem_g[(u + 1) % RR])

                if with_norm:
                    @pl.when((j >= 4) & (j <= NCHUNK + 3))
                    def _():
                        off = base + (j - 4) * CH
                        pltpu.make_async_copy(
                            nbuf[ti], norm_out.at[pl.ds(off, CH)],
                            sem_n[ti]).wait()

                    @pl.when(j <= NCHUNK - 1)
                    def _():
                        for g in range(CH // 16):
                            sl = pl.ds(g * 16, 16)
                            dr = plsc.load_gather(dinv_v, [ridx[ti][sl]])
                            dc = plsc.load_gather(dinv_v, [cidx[ti][sl]])
                            nbuf[ti][sl] = dr * wbuf[ti][sl] * dc
                        pltpu.async_copy(
                            nbuf[ti],
                            norm_out.at[pl.ds(base + j * CH, CH)],
                            sem_n[ti])

                @pl.when(j <= NCHUNK - 1)
                def _():
                    pltpu.make_async_copy(
                        xw_hbm.at[ridx[ti]], rows[tr], sem_g[tr]).wait()

                    def sc(k, _):
                        bc = plsc.load_gather(
                            nbuf[ti], [jnp.full((16,), k, jnp.int32)])
                        for f in range(nfeat // 16):
                            sl = pl.ds(f * 16, 16)
                            rows[tr][k, sl] = rows[tr][k, sl] * bc
                        return 0

                    lax.fori_loop(0, CH, sc, 0)
                    pltpu.async_copy(rows[tr], out_sh.at[cidx[ti]],
                                     sem_s[tr], add=True)
            return 0

        lax.fori_loop(0, (NCHUNK + 7 + UNROLL - 1) // UNROLL, group, 0)
        plsc.subcore_barrier()
        pltpu.sync_copy(out_sh.at[pl.ds(s * RPT, RPT)],
                        part_out.at[c, pl.ds(s * RPT, RPT)])

    return pl.kernel(
        body,
        out_type=outs if with_norm else outs[0],
        mesh=plsc.VectorSubcoreMesh(**_MESH),
        scratch_types=scratch,
        compiler_params=pltpu.CompilerParams(
            needs_layout_passes=False,
            disable_bounds_checks=True,
            use_tc_tiling_on_sc=None if nfeat % 128 == 0 else False,
        ),
    )


_prop1 = _make_prop_kernel(F1, with_norm=True)
_prop2 = _make_prop_kernel(F2, with_norm=False, chunk=200)


def _matmul1_body(x_ref, w_ref, o_ref):
    o_ref[...] = lax.dot_general(
        x_ref[...], w_ref[...], (((1,), (1,)), ((), ())),
        preferred_element_type=jnp.float32)


def _dinv_body(deg_ref, dinv_ref, dinv2_ref):
    deg = deg_ref[0, :] + deg_ref[1, :] + 1.0
    dinv = jnp.where(deg > 0, lax.rsqrt(deg), 0.0)
    dinv_ref[...] = dinv
    dinv2_ref[...] = dinv * dinv


def _mid_body(p0_ref, p1_ref, xw_ref, dinv2_ref, b_ref, w2_ref, o_ref):
    h = (p0_ref[0] + p1_ref[0] + dinv2_ref[...] * xw_ref[...] + b_ref[...])
    h = jnp.maximum(h, 0.0)
    o_ref[...] = lax.dot_general(
        h, w2_ref[...], (((1,), (1,)), ((), ())),
        preferred_element_type=jnp.float32)


def _final_body(p0_ref, p1_ref, xw_ref, dinv2_ref, b_ref, o_ref):
    t = (p0_ref[0] + p1_ref[0] + dinv2_ref[...] * xw_ref[...] + b_ref[...])
    m = jnp.max(t, axis=1, keepdims=True)
    lse = jnp.log(jnp.sum(jnp.exp(t - m), axis=1, keepdims=True)) + m
    o_ref[...] = t - lse


_ROWB = 1000


def kernel(x, edge_index, edge_weight, W1, b1, W2, b2):
    row = edge_index[0].astype(jnp.int32)
    col = edge_index[1].astype(jnp.int32)
    w = edge_weight.astype(jnp.float32)

    deg_p = _deg_kernel(col, w)

    xw1 = pl.pallas_call(
        _matmul1_body,
        grid=(N // _ROWB,),
        in_specs=[pl.BlockSpec((_ROWB, F1), lambda i: (i, 0)),
                  pl.BlockSpec((F1, F1), lambda i: (0, 0))],
        out_specs=pl.BlockSpec((_ROWB, F1), lambda i: (i, 0)),
        out_shape=jax.ShapeDtypeStruct((N, F1), jnp.float32),
    )(x, W1)

    dinv, dinv2 = pl.pallas_call(
        _dinv_body,
        out_shape=[jax.ShapeDtypeStruct((NPAD,), jnp.float32),
                   jax.ShapeDtypeStruct((NPAD,), jnp.float32)],
    )(deg_p)

    norm, part1 = _prop1(row, col, w, dinv, xw1)

    dinv2c = dinv2[:N, None]
    xw2 = pl.pallas_call(
        _mid_body,
        grid=(N // _ROWB,),
        in_specs=[pl.BlockSpec((1, _ROWB, F1), lambda i: (0, i, 0)),
                  pl.BlockSpec((1, _ROWB, F1), lambda i: (1, i, 0)),
                  pl.BlockSpec((_ROWB, F1), lambda i: (i, 0)),
                  pl.BlockSpec((_ROWB, 1), lambda i: (i, 0)),
                  pl.BlockSpec((1, F1), lambda i: (0, 0)),
                  pl.BlockSpec((F2, F1), lambda i: (0, 0))],
        out_specs=pl.BlockSpec((_ROWB, F2), lambda i: (i, 0)),
        out_shape=jax.ShapeDtypeStruct((N, F2), jnp.float32),
    )(part1, part1, xw1, dinv2c, b1[None, :], W2)

    part2 = _prop2(row, col, norm, xw2)

    out = pl.pallas_call(
        _final_body,
        grid=(N // _ROWB,),
        in_specs=[pl.BlockSpec((1, _ROWB, F2), lambda i: (0, i, 0)),
                  pl.BlockSpec((1, _ROWB, F2), lambda i: (1, i, 0)),
                  pl.BlockSpec((_ROWB, F2), lambda i: (i, 0)),
                  pl.BlockSpec((_ROWB, 1), lambda i: (i, 0)),
                  pl.BlockSpec((1, F2), lambda i: (0, 0))],
        out_specs=pl.BlockSpec((_ROWB, F2), lambda i: (i, 0)),
        out_shape=jax.ShapeDtypeStruct((N, F2), jnp.float32),
    )(part2, part2, xw2, dinv2c, b2[None, :])

    return out


# speedup vs baseline: 1.1381x; 1.0269x over previous
import functools

import jax
import jax.numpy as jnp
from jax import lax
from jax.experimental import pallas as pl
from jax.experimental.pallas import tpu as pltpu
from jax.experimental.pallas import tpu_sc as plsc

N = 10000
NPAD = 10240
E = 320000
F1 = 128
F2 = 64
NW = 32
EPT = E // NW
CH = 80
NCHUNK = EPT // CH
DCH = 200
DNCHUNK = EPT // DCH
RPT = NPAD // 16

RI = 4
RR = 3
UNROLL = 12

_MESH = dict(core_axis_name="c", subcore_axis_name="s")


def _wid():
    return lax.axis_index("s") * 2 + lax.axis_index("c")


def _zero16(buf, nwords):
    def zb(i, _):
        buf[pl.ds(i * 16, 16)] = jnp.zeros((16,), jnp.float32)
        return 0
    lax.fori_loop(0, nwords, zb, 0)


@functools.partial(
    pl.kernel,
    out_type=jax.ShapeDtypeStruct((2, NPAD), jnp.float32),
    mesh=plsc.VectorSubcoreMesh(**_MESH),
    scratch_types=(
        [pltpu.VMEM((DCH,), jnp.int32) for _ in range(RI)]
        + [pltpu.VMEM((DCH,), jnp.float32) for _ in range(RI)]
        + [pltpu.VMEM((RPT,), jnp.float32),
           pltpu.VMEM_SHARED((NPAD,), jnp.float32)]
        + [pltpu.SemaphoreType.DMA for _ in range(2 * RI)]
    ),
    compiler_params=pltpu.CompilerParams(
        needs_layout_passes=False, disable_bounds_checks=True),
)
def _deg_kernel(col_hbm, w_hbm, deg_out, *sc):
    cbuf, wbuf = list(sc[0:RI]), list(sc[RI:2 * RI])
    zbuf, deg_sh = sc[2 * RI], sc[2 * RI + 1]
    sem_i = list(sc[2 * RI + 2:2 * RI + 2 + RI])
    sem_s = list(sc[2 * RI + 2 + RI:])
    c = lax.axis_index("c")
    s = lax.axis_index("s")
    base = _wid() * EPT

    def issue_idx(t, ch):
        off = base + ch * DCH
        pltpu.async_copy(col_hbm.at[pl.ds(off, DCH)], cbuf[t], sem_i[t])
        pltpu.async_copy(w_hbm.at[pl.ds(off, DCH)], wbuf[t], sem_i[t])

    def wait_idx(t, ch):
        off = base + ch * DCH
        pltpu.make_async_copy(
            col_hbm.at[pl.ds(off, DCH)], cbuf[t], sem_i[t]).wait()
        pltpu.make_async_copy(
            w_hbm.at[pl.ds(off, DCH)], wbuf[t], sem_i[t]).wait()

    issue_idx(0, 0)
    issue_idx(1, 1)
    _zero16(zbuf, RPT // 16)
    pltpu.sync_copy(zbuf, deg_sh.at[pl.ds(s * RPT, RPT)])
    plsc.subcore_barrier()

    def group(h, _):
        for u in range(4):
            j = h * 4 + u
            t = u % 4

            @pl.when((j >= 2) & (j <= DNCHUNK + 1))
            def _():
                t2 = (u - 2) % 4
                pltpu.make_async_copy(
                    wbuf[t2], deg_sh.at[cbuf[t2]], sem_s[t2]).wait()

            @pl.when(j <= DNCHUNK - 3)
            def _():
                issue_idx((u + 2) % 4, j + 2)

            @pl.when(j <= DNCHUNK - 1)
            def _():
                wait_idx(t, j)
                pltpu.async_copy(wbuf[t], deg_sh.at[cbuf[t]], sem_s[t],
                                 add=True)
        return 0

    lax.fori_loop(0, 32, group, 0)
    plsc.subcore_barrier()
    pltpu.sync_copy(deg_sh.at[pl.ds(s * RPT, RPT)],
                    deg_out.at[c, pl.ds(s * RPT, RPT)])


def _make_prop_kernel(nfeat, with_norm, chunk=CH):
    CH = chunk
    NCHUNK = EPT // chunk
    outs = [jax.ShapeDtypeStruct((2, NPAD, nfeat), jnp.float32)]
    if with_norm:
        outs = [jax.ShapeDtypeStruct((E,), jnp.float32)] + outs
    ZR = 40
    scratch = (
        [pltpu.VMEM((CH,), jnp.int32) for _ in range(RI)]
        + [pltpu.VMEM((CH,), jnp.int32) for _ in range(RI)]
        + [pltpu.VMEM((CH,), jnp.float32) for _ in range(RI)]
        + [pltpu.VMEM((CH, nfeat), jnp.float32) for _ in range(RR)]
        + [pltpu.VMEM((ZR, nfeat), jnp.float32),
           pltpu.VMEM_SHARED((NPAD, nfeat), jnp.float32)]
        + [pltpu.SemaphoreType.DMA for _ in range(RI + RR + RR)]
    )
    if with_norm:
        scratch = (
            [pltpu.VMEM((CH,), jnp.float32) for _ in range(RI)]
            + [pltpu.VMEM((NPAD,), jnp.float32)]
            + scratch
            + [pltpu.SemaphoreType.DMA for _ in range(RI)]
        )

    def body(*refs):
        if with_norm:
            (row_hbm, col_hbm, w_hbm, dinv_hbm, xw_hbm,
             norm_out, part_out) = refs[:7]
            wbuf = list(refs[7:7 + RI])
            dinv_v = refs[7 + RI]
            rest = refs[8 + RI:]
        else:
            (row_hbm, col_hbm, norm_hbm, xw_hbm, part_out) = refs[:5]
            rest = refs[5:]
        ridx = list(rest[0:RI])
        cidx = list(rest[RI:2 * RI])
        nbuf = list(rest[2 * RI:3 * RI])
        rows = list(rest[3 * RI:3 * RI + RR])
        zbuf = rest[3 * RI + RR]
        out_sh = rest[3 * RI + RR + 1]
        sems = rest[3 * RI + RR + 2:]
        sem_i = list(sems[0:RI])
        sem_g = list(sems[RI:RI + RR])
        sem_s = list(sems[RI + RR:RI + RR + RR])
        if with_norm:
            sem_n = list(sems[RI + RR + RR:])

        c = lax.axis_index("c")
        s = lax.axis_index("s")
        base = _wid() * EPT

        def issue_idx(t, ch):
            off = base + ch * CH
            pltpu.async_copy(row_hbm.at[pl.ds(off, CH)], ridx[t], sem_i[t])
            pltpu.async_copy(col_hbm.at[pl.ds(off, CH)], cidx[t], sem_i[t])
            if with_norm:
                pltpu.async_copy(w_hbm.at[pl.ds(off, CH)], wbuf[t], sem_i[t])
            else:
                pltpu.async_copy(norm_hbm.at[pl.ds(off, CH)], nbuf[t],
                                 sem_i[t])

        def wait_idx(t, ch):
            off = base + ch * CH
            pltpu.make_async_copy(
                row_hbm.at[pl.ds(off, CH)], ridx[t], sem_i[t]).wait()
            pltpu.make_async_copy(
                col_hbm.at[pl.ds(off, CH)], cidx[t], sem_i[t]).wait()
            if with_norm:
                pltpu.make_async_copy(
                    w_hbm.at[pl.ds(off, CH)], wbuf[t], sem_i[t]).wait()
            else:
                pltpu.make_async_copy(
                    norm_hbm.at[pl.ds(off, CH)], nbuf[t], sem_i[t]).wait()

        issue_idx(0, 0)
        issue_idx(1, 1)
        if with_norm:
            pltpu.async_copy(dinv_hbm, dinv_v, sem_g[1])

        def zz(i, _):
            for f in range(nfeat // 16):
                zbuf[i, pl.ds(f * 16, 16)] = jnp.zeros((16,), jnp.float32)
            return 0

        lax.fori_loop(0, ZR, zz, 0)
        nz = RPT // ZR
        for q in range(nz):
            pltpu.async_copy(zbuf, out_sh.at[pl.ds(s * RPT + q * ZR, ZR)],
                             sem_s[q % RR])
        wait_idx(0, 0)
        if with_norm:
            pltpu.make_async_copy(dinv_hbm, dinv_v, sem_g[1]).wait()
        pltpu.async_copy(xw_hbm.at[ridx[0]], rows[0], sem_g[0])
        for q in range(nz):
            pltpu.make_async_copy(
                zbuf, out_sh.at[pl.ds(s * RPT + q * ZR, ZR)],
                sem_s[q % RR]).wait()
        plsc.subcore_barrier()

        def group(h, _):
            for u in range(UNROLL):
                j = h * UNROLL + u
                ti = u % RI
                tr = u % RR

                @pl.when((j >= 2) & (j <= NCHUNK + 1))
                def _():
                    pltpu.make_async_copy(
                        rows[(u - 2) % RR],
                        out_sh.at[cidx[(u - 2) % RI]],
                        sem_s[(u - 2) % RR]).wait()

                @pl.when(j <= NCHUNK - 3)
                def _():
                    issue_idx((u + 2) % RI, j + 2)

                @pl.when((j >= 0) & (j <= NCHUNK - 2))
                def _():
                    wait_idx((u + 1) % RI, j + 1)
                    pltpu.async_copy(xw_hbm.at[ridx[(u + 1) % RI]],
                                     rows[(u + 1) % RR],
                                     sem_g[(u + 1) % RR])

                if with_norm:
                    @pl.when((j >= 4) & (j <= NCHUNK + 3))
                    def _():
                        off = base + (j - 4) * CH
                        pltpu.make_async_copy(
                            nbuf[ti], norm_out.at[pl.ds(off, CH)],
                            sem_n[ti]).wait()

                    @pl.when(j <= NCHUNK - 1)
                    def _():
                        for g in range(CH // 16):
                            sl = pl.ds(g * 16, 16)
                            dr = plsc.load_gather(dinv_v, [ridx[ti][sl]])
                            dc = plsc.load_gather(dinv_v, [cidx[ti][sl]])
                            nbuf[ti][sl] = dr * wbuf[ti][sl] * dc
                        pltpu.async_copy(
                            nbuf[ti],
                            norm_out.at[pl.ds(base + j * CH, CH)],
                            sem_n[ti])

                @pl.when(j <= NCHUNK - 1)
                def _():
                    pltpu.make_async_copy(
                        xw_hbm.at[ridx[ti]], rows[tr], sem_g[tr]).wait()

                    def sc(k, _):
                        bc = plsc.load_gather(
                            nbuf[ti], [jnp.full((16,), k, jnp.int32)])
                        for f in range(nfeat // 16):
                            sl = pl.ds(f * 16, 16)
                            rows[tr][k, sl] = rows[tr][k, sl] * bc
                        return 0

                    lax.fori_loop(0, CH, sc, 0)
                    pltpu.async_copy(rows[tr], out_sh.at[cidx[ti]],
                                     sem_s[tr], add=True)
            return 0

        lax.fori_loop(0, (NCHUNK + 7 + UNROLL - 1) // UNROLL, group, 0)
        plsc.subcore_barrier()
        pltpu.sync_copy(out_sh.at[pl.ds(s * RPT, RPT)],
                        part_out.at[c, pl.ds(s * RPT, RPT)])

    return pl.kernel(
        body,
        out_type=outs if with_norm else outs[0],
        mesh=plsc.VectorSubcoreMesh(**_MESH),
        scratch_types=scratch,
        compiler_params=pltpu.CompilerParams(
            needs_layout_passes=False,
            disable_bounds_checks=True,
            use_tc_tiling_on_sc=None if nfeat % 128 == 0 else False,
        ),
    )


_prop1 = _make_prop_kernel(F1, with_norm=True)
_prop2 = _make_prop_kernel(F2, with_norm=False, chunk=400)


def _matmul1_body(x_ref, w_ref, o_ref):
    o_ref[...] = lax.dot_general(
        x_ref[...], w_ref[...], (((1,), (1,)), ((), ())),
        preferred_element_type=jnp.float32)


def _dinv_body(deg_ref, dinv_ref, dinv2_ref):
    deg = deg_ref[0, :] + deg_ref[1, :] + 1.0
    dinv = jnp.where(deg > 0, lax.rsqrt(deg), 0.0)
    dinv_ref[...] = dinv
    dinv2_ref[...] = dinv * dinv


def _mid_body(p0_ref, p1_ref, xw_ref, dinv2_ref, b_ref, w2_ref, o_ref):
    h = (p0_ref[0] + p1_ref[0] + dinv2_ref[...] * xw_ref[...] + b_ref[...])
    h = jnp.maximum(h, 0.0)
    o_ref[...] = lax.dot_general(
        h, w2_ref[...], (((1,), (1,)), ((), ())),
        preferred_element_type=jnp.float32)


def _final_body(p0_ref, p1_ref, xw_ref, dinv2_ref, b_ref, o_ref):
    t = (p0_ref[0] + p1_ref[0] + dinv2_ref[...] * xw_ref[...] + b_ref[...])
    m = jnp.max(t, axis=1, keepdims=True)
    lse = jnp.log(jnp.sum(jnp.exp(t - m), axis=1, keepdims=True)) + m
    o_ref[...] = t - lse


_ROWB = 1000


def kernel(x, edge_index, edge_weight, W1, b1, W2, b2):
    row = edge_index[0].astype(jnp.int32)
    col = edge_index[1].astype(jnp.int32)
    w = edge_weight.astype(jnp.float32)

    deg_p = _deg_kernel(col, w)

    xw1 = pl.pallas_call(
        _matmul1_body,
        grid=(N // _ROWB,),
        in_specs=[pl.BlockSpec((_ROWB, F1), lambda i: (i, 0)),
                  pl.BlockSpec((F1, F1), lambda i: (0, 0))],
        out_specs=pl.BlockSpec((_ROWB, F1), lambda i: (i, 0)),
        out_shape=jax.ShapeDtypeStruct((N, F1), jnp.float32),
    )(x, W1)

    dinv, dinv2 = pl.pallas_call(
        _dinv_body,
        out_shape=[jax.ShapeDtypeStruct((NPAD,), jnp.float32),
                   jax.ShapeDtypeStruct((NPAD,), jnp.float32)],
    )(deg_p)

    norm, part1 = _prop1(row, col, w, dinv, xw1)

    dinv2c = dinv2[:N, None]
    xw2 = pl.pallas_call(
        _mid_body,
        grid=(N // _ROWB,),
        in_specs=[pl.BlockSpec((1, _ROWB, F1), lambda i: (0, i, 0)),
                  pl.BlockSpec((1, _ROWB, F1), lambda i: (1, i, 0)),
                  pl.BlockSpec((_ROWB, F1), lambda i: (i, 0)),
                  pl.BlockSpec((_ROWB, 1), lambda i: (i, 0)),
                  pl.BlockSpec((1, F1), lambda i: (0, 0)),
                  pl.BlockSpec((F2, F1), lambda i: (0, 0))],
        out_specs=pl.BlockSpec((_ROWB, F2), lambda i: (i, 0)),
        out_shape=jax.ShapeDtypeStruct((N, F2), jnp.float32),
    )(part1, part1, xw1, dinv2c, b1[None, :], W2)

    part2 = _prop2(row, col, norm, xw2)

    out = pl.pallas_call(
        _final_body,
        grid=(N // _ROWB,),
        in_specs=[pl.BlockSpec((1, _ROWB, F2), lambda i: (0, i, 0)),
                  pl.BlockSpec((1, _ROWB, F2), lambda i: (1, i, 0)),
                  pl.BlockSpec((_ROWB, F2), lambda i: (i, 0)),
                  pl.BlockSpec((_ROWB, 1), lambda i: (i, 0)),
                  pl.BlockSpec((1, F2), lambda i: (0, 0))],
        out_specs=pl.BlockSpec((_ROWB, F2), lambda i: (i, 0)),
        out_shape=jax.ShapeDtypeStruct((N, F2), jnp.float32),
    )(part2, part2, xw2, dinv2c, b2[None, :])

    return out


# speedup vs baseline: 1.1512x; 1.0115x over previous
import functools

import jax
import jax.numpy as jnp
from jax import lax
from jax.experimental import pallas as pl
from jax.experimental.pallas import tpu as pltpu
from jax.experimental.pallas import tpu_sc as plsc

N = 10000
NPAD = 10240
E = 320000
F1 = 128
F2 = 64
NW = 32
EPT = E // NW
CH = 80
NCHUNK = EPT // CH
DCH = 400
DNCHUNK = EPT // DCH
RPT = NPAD // 16

RI = 4
RR = 3
UNROLL = 12

_MESH = dict(core_axis_name="c", subcore_axis_name="s")


def _wid():
    return lax.axis_index("s") * 2 + lax.axis_index("c")


def _zero16(buf, nwords):
    def zb(i, _):
        buf[pl.ds(i * 16, 16)] = jnp.zeros((16,), jnp.float32)
        return 0
    lax.fori_loop(0, nwords, zb, 0)


@functools.partial(
    pl.kernel,
    out_type=jax.ShapeDtypeStruct((2, NPAD), jnp.float32),
    mesh=plsc.VectorSubcoreMesh(**_MESH),
    scratch_types=(
        [pltpu.VMEM((DCH,), jnp.int32) for _ in range(RI)]
        + [pltpu.VMEM((DCH,), jnp.float32) for _ in range(RI)]
        + [pltpu.VMEM((RPT,), jnp.float32),
           pltpu.VMEM_SHARED((NPAD,), jnp.float32)]
        + [pltpu.SemaphoreType.DMA for _ in range(2 * RI)]
    ),
    compiler_params=pltpu.CompilerParams(
        needs_layout_passes=False, disable_bounds_checks=True),
)
def _deg_kernel(col_hbm, w_hbm, deg_out, *sc):
    cbuf, wbuf = list(sc[0:RI]), list(sc[RI:2 * RI])
    zbuf, deg_sh = sc[2 * RI], sc[2 * RI + 1]
    sem_i = list(sc[2 * RI + 2:2 * RI + 2 + RI])
    sem_s = list(sc[2 * RI + 2 + RI:])
    c = lax.axis_index("c")
    s = lax.axis_index("s")
    base = _wid() * EPT

    def issue_idx(t, ch):
        off = base + ch * DCH
        pltpu.async_copy(col_hbm.at[pl.ds(off, DCH)], cbuf[t], sem_i[t])
        pltpu.async_copy(w_hbm.at[pl.ds(off, DCH)], wbuf[t], sem_i[t])

    def wait_idx(t, ch):
        off = base + ch * DCH
        pltpu.make_async_copy(
            col_hbm.at[pl.ds(off, DCH)], cbuf[t], sem_i[t]).wait()
        pltpu.make_async_copy(
            w_hbm.at[pl.ds(off, DCH)], wbuf[t], sem_i[t]).wait()

    issue_idx(0, 0)
    issue_idx(1, 1)
    _zero16(zbuf, RPT // 16)
    pltpu.sync_copy(zbuf, deg_sh.at[pl.ds(s * RPT, RPT)])
    plsc.subcore_barrier()

    def group(h, _):
        for u in range(4):
            j = h * 4 + u
            t = u % 4

            @pl.when((j >= 2) & (j <= DNCHUNK + 1))
            def _():
                t2 = (u - 2) % 4
                pltpu.make_async_copy(
                    wbuf[t2], deg_sh.at[cbuf[t2]], sem_s[t2]).wait()

            @pl.when(j <= DNCHUNK - 3)
            def _():
                issue_idx((u + 2) % 4, j + 2)

            @pl.when(j <= DNCHUNK - 1)
            def _():
                wait_idx(t, j)
                pltpu.async_copy(wbuf[t], deg_sh.at[cbuf[t]], sem_s[t],
                                 add=True)
        return 0

    lax.fori_loop(0, 32, group, 0)
    plsc.subcore_barrier()
    pltpu.sync_copy(deg_sh.at[pl.ds(s * RPT, RPT)],
                    deg_out.at[c, pl.ds(s * RPT, RPT)])


def _make_prop_kernel(nfeat, with_norm, chunk=CH):
    CH = chunk
    NCHUNK = EPT // chunk
    outs = [jax.ShapeDtypeStruct((2, NPAD, nfeat), jnp.float32)]
    if with_norm:
        outs = [jax.ShapeDtypeStruct((E,), jnp.float32)] + outs
    ZR = 40
    scratch = (
        [pltpu.VMEM((CH,), jnp.int32) for _ in range(RI)]
        + [pltpu.VMEM((CH,), jnp.int32) for _ in range(RI)]
        + [pltpu.VMEM((CH,), jnp.float32) for _ in range(RI)]
        + [pltpu.VMEM((CH, nfeat), jnp.float32) for _ in range(RR)]
        + [pltpu.VMEM((ZR, nfeat), jnp.float32),
           pltpu.VMEM_SHARED((NPAD, nfeat), jnp.float32)]
        + [pltpu.SemaphoreType.DMA for _ in range(RI + RR + RR)]
    )
    if with_norm:
        scratch = (
            [pltpu.VMEM((CH,), jnp.float32) for _ in range(RI)]
            + [pltpu.VMEM((NPAD,), jnp.float32)]
            + scratch
            + [pltpu.SemaphoreType.DMA for _ in range(RI)]
        )

    def body(*refs):
        if with_norm:
            (row_hbm, col_hbm, w_hbm, dinv_hbm, xw_hbm,
             norm_out, part_out) = refs[:7]
            wbuf = list(refs[7:7 + RI])
            dinv_v = refs[7 + RI]
            rest = refs[8 + RI:]
        else:
            (row_hbm, col_hbm, norm_hbm, xw_hbm, part_out) = refs[:5]
            rest = refs[5:]
        ridx = list(rest[0:RI])
        cidx = list(rest[RI:2 * RI])
        nbuf = list(rest[2 * RI:3 * RI])
        rows = list(rest[3 * RI:3 * RI + RR])
        zbuf = rest[3 * RI + RR]
        out_sh = rest[3 * RI + RR + 1]
        sems = rest[3 * RI + RR + 2:]
        sem_i = list(sems[0:RI])
        sem_g = list(sems[RI:RI + RR])
        sem_s = list(sems[RI + RR:RI + RR + RR])
        if with_norm:
            sem_n = list(sems[RI + RR + RR:])

        c = lax.axis_index("c")
        s = lax.axis_index("s")
        base = _wid() * EPT

        def issue_idx(t, ch):
            off = base + ch * CH
            pltpu.async_copy(row_hbm.at[pl.ds(off, CH)], ridx[t], sem_i[t])
            pltpu.async_copy(col_hbm.at[pl.ds(off, CH)], cidx[t], sem_i[t])
            if with_norm:
                pltpu.async_copy(w_hbm.at[pl.ds(off, CH)], wbuf[t], sem_i[t])
            else:
                pltpu.async_copy(norm_hbm.at[pl.ds(off, CH)], nbuf[t],
                                 sem_i[t])

        def wait_idx(t, ch):
            off = base + ch * CH
            pltpu.make_async_copy(
                row_hbm.at[pl.ds(off, CH)], ridx[t], sem_i[t]).wait()
            pltpu.make_async_copy(
                col_hbm.at[pl.ds(off, CH)], cidx[t], sem_i[t]).wait()
            if with_norm:
                pltpu.make_async_copy(
                    w_hbm.at[pl.ds(off, CH)], wbuf[t], sem_i[t]).wait()
            else:
                pltpu.make_async_copy(
                    norm_hbm.at[pl.ds(off, CH)], nbuf[t], sem_i[t]).wait()

        issue_idx(0, 0)
        issue_idx(1, 1)
        if with_norm:
            pltpu.async_copy(dinv_hbm, dinv_v, sem_g[1])

        def zz(i, _):
            for f in range(nfeat // 16):
                zbuf[i, pl.ds(f * 16, 16)] = jnp.zeros((16,), jnp.float32)
            return 0

        lax.fori_loop(0, ZR, zz, 0)
        nz = RPT // ZR
        for q in range(nz):
            pltpu.async_copy(zbuf, out_sh.at[pl.ds(s * RPT + q * ZR, ZR)],
                             sem_s[q % RR])
        wait_idx(0, 0)
        if with_norm:
            pltpu.make_async_copy(dinv_hbm, dinv_v, sem_g[1]).wait()
        pltpu.async_copy(xw_hbm.at[ridx[0]], rows[0], sem_g[0])
        for q in range(nz):
            pltpu.make_async_copy(
                zbuf, out_sh.at[pl.ds(s * RPT + q * ZR, ZR)],
                sem_s[q % RR]).wait()
        plsc.subcore_barrier()

        def group(h, _):
            for u in range(UNROLL):
                j = h * UNROLL + u
                ti = u % RI
                tr = u % RR

                @pl.when((j >= 2) & (j <= NCHUNK + 1))
                def _():
                    pltpu.make_async_copy(
                        rows[(u - 2) % RR],
                        out_sh.at[cidx[(u - 2) % RI]],
                        sem_s[(u - 2) % RR]).wait()

                @pl.when(j <= NCHUNK - 3)
                def _():
                    issue_idx((u + 2) % RI, j + 2)

                @pl.when((j >= 0) & (j <= NCHUNK - 2))
                def _():
                    wait_idx((u + 1) % RI, j + 1)
                    pltpu.async_copy(xw_hbm.at[ridx[(u + 1) % RI]],
                                     rows[(u + 1) % RR],
                                     sem_g[(u + 1) % RR])

                if with_norm:
                    @pl.when((j >= 4) & (j <= NCHUNK + 3))
                    def _():
                        off = base + (j - 4) * CH
                        pltpu.make_async_copy(
                            nbuf[ti], norm_out.at[pl.ds(off, CH)],
                            sem_n[ti]).wait()

                    @pl.when(j <= NCHUNK - 1)
                    def _():
                        for g in range(CH // 16):
                            sl = pl.ds(g * 16, 16)
                            dr = plsc.load_gather(dinv_v, [ridx[ti][sl]])
                            dc = plsc.load_gather(dinv_v, [cidx[ti][sl]])
                            nbuf[ti][sl] = dr * wbuf[ti][sl] * dc
                        pltpu.async_copy(
                            nbuf[ti],
                            norm_out.at[pl.ds(base + j * CH, CH)],
                            sem_n[ti])

                @pl.when(j <= NCHUNK - 1)
                def _():
                    pltpu.make_async_copy(
                        xw_hbm.at[ridx[ti]], rows[tr], sem_g[tr]).wait()

                    def sc(k, _):
                        bc = plsc.load_gather(
                            nbuf[ti], [jnp.full((16,), k, jnp.int32)])
                        for f in range(nfeat // 16):
                            sl = pl.ds(f * 16, 16)
                            rows[tr][k, sl] = rows[tr][k, sl] * bc
                        return 0

                    lax.fori_loop(0, CH, sc, 0)
                    pltpu.async_copy(rows[tr], out_sh.at[cidx[ti]],
                                     sem_s[tr], add=True)
            return 0

        lax.fori_loop(0, (NCHUNK + 7 + UNROLL - 1) // UNROLL, group, 0)
        plsc.subcore_barrier()
        pltpu.sync_copy(out_sh.at[pl.ds(s * RPT, RPT)],
                        part_out.at[c, pl.ds(s * RPT, RPT)])

    return pl.kernel(
        body,
        out_type=outs if with_norm else outs[0],
        mesh=plsc.VectorSubcoreMesh(**_MESH),
        scratch_types=scratch,
        compiler_params=pltpu.CompilerParams(
            needs_layout_passes=False,
            disable_bounds_checks=True,
            use_tc_tiling_on_sc=None if nfeat % 128 == 0 else False,
        ),
    )


_prop1 = _make_prop_kernel(F1, with_norm=True)
_prop2 = _make_prop_kernel(F2, with_norm=False, chunk=400)


def _matmul1_body(x_ref, w_ref, o_ref):
    o_ref[...] = lax.dot_general(
        x_ref[...], w_ref[...], (((1,), (1,)), ((), ())),
        preferred_element_type=jnp.float32)


def _dinv_body(deg_ref, dinv_ref, dinv2_ref):
    deg = deg_ref[0, :] + deg_ref[1, :] + 1.0
    dinv = jnp.where(deg > 0, lax.rsqrt(deg), 0.0)
    dinv_ref[...] = dinv
    dinv2_ref[...] = dinv * dinv


def _mid_body(p0_ref, p1_ref, xw_ref, dinv2_ref, b_ref, w2_ref, o_ref):
    h = (p0_ref[0] + p1_ref[0] + dinv2_ref[...] * xw_ref[...] + b_ref[...])
    h = jnp.maximum(h, 0.0)
    o_ref[...] = lax.dot_general(
        h, w2_ref[...], (((1,), (1,)), ((), ())),
        preferred_element_type=jnp.float32)


def _final_body(p0_ref, p1_ref, xw_ref, dinv2_ref, b_ref, o_ref):
    t = (p0_ref[0] + p1_ref[0] + dinv2_ref[...] * xw_ref[...] + b_ref[...])
    m = jnp.max(t, axis=1, keepdims=True)
    lse = jnp.log(jnp.sum(jnp.exp(t - m), axis=1, keepdims=True)) + m
    o_ref[...] = t - lse


_ROWB = 1000


def kernel(x, edge_index, edge_weight, W1, b1, W2, b2):
    row = edge_index[0].astype(jnp.int32)
    col = edge_index[1].astype(jnp.int32)
    w = edge_weight.astype(jnp.float32)

    deg_p = _deg_kernel(col, w)

    xw1 = pl.pallas_call(
        _matmul1_body,
        grid=(N // _ROWB,),
        in_specs=[pl.BlockSpec((_ROWB, F1), lambda i: (i, 0)),
                  pl.BlockSpec((F1, F1), lambda i: (0, 0))],
        out_specs=pl.BlockSpec((_ROWB, F1), lambda i: (i, 0)),
        out_shape=jax.ShapeDtypeStruct((N, F1), jnp.float32),
    )(x, W1)

    dinv, dinv2 = pl.pallas_call(
        _dinv_body,
        out_shape=[jax.ShapeDtypeStruct((NPAD,), jnp.float32),
                   jax.ShapeDtypeStruct((NPAD,), jnp.float32)],
    )(deg_p)

    norm, part1 = _prop1(row, col, w, dinv, xw1)

    dinv2c = dinv2[:N, None]
    xw2 = pl.pallas_call(
        _mid_body,
        grid=(N // _ROWB,),
        in_specs=[pl.BlockSpec((1, _ROWB, F1), lambda i: (0, i, 0)),
                  pl.BlockSpec((1, _ROWB, F1), lambda i: (1, i, 0)),
                  pl.BlockSpec((_ROWB, F1), lambda i: (i, 0)),
                  pl.BlockSpec((_ROWB, 1), lambda i: (i, 0)),
                  pl.BlockSpec((1, F1), lambda i: (0, 0)),
                  pl.BlockSpec((F2, F1), lambda i: (0, 0))],
        out_specs=pl.BlockSpec((_ROWB, F2), lambda i: (i, 0)),
        out_shape=jax.ShapeDtypeStruct((N, F2), jnp.float32),
    )(part1, part1, xw1, dinv2c, b1[None, :], W2)

    part2 = _prop2(row, col, norm, xw2)

    out = pl.pallas_call(
        _final_body,
        grid=(N // _ROWB,),
        in_specs=[pl.BlockSpec((1, _ROWB, F2), lambda i: (0, i, 0)),
                  pl.BlockSpec((1, _ROWB, F2), lambda i: (1, i, 0)),
                  pl.BlockSpec((_ROWB, F2), lambda i: (i, 0)),
                  pl.BlockSpec((_ROWB, 1), lambda i: (i, 0)),
                  pl.BlockSpec((1, F2), lambda i: (0, 0))],
        out_specs=pl.BlockSpec((_ROWB, F2), lambda i: (i, 0)),
        out_shape=jax.ShapeDtypeStruct((N, F2), jnp.float32),
    )(part2, part2, xw2, dinv2c, b2[None, :])

    return out
